# baseline (speedup 1.0000x reference)
import numpy as np
import jax
import jax.numpy as jnp
from jax import lax
from jax.experimental import pallas as pl
from jax.experimental.pallas import tpu as pltpu
from jax.experimental.pallas import tpu_sc as plsc

N = 262144
D = 119
ROWS = 2048

N_SC = 16384
N_TC = N - N_SC
SC_CORES = 2
SC_SUBCORES = 16
SC_WORKERS = SC_CORES * SC_SUBCORES
RW = N_SC // SC_WORKERS
BR = 256
SC_BLOCKS = RW // BR

K_SAMP = (1832780943, 270669613)
K_JUMP = (64467757, 2916123636)
TINY = float(np.finfo(np.float32).tiny)
MAGIC = 0x1BD11BDA

_ROT_A = (13, 15, 26, 6)
_ROT_B = (17, 29, 16, 24)


def _threefry_xor(key, ctr):
    k0, k1 = int(key[0]), int(key[1])
    ks = (k0, k1, k0 ^ k1 ^ MAGIC)
    x0 = jnp.full(ctr.shape, jnp.uint32(ks[0]), jnp.uint32)
    x1 = ctr + jnp.uint32(ks[1])
    for i in range(5):
        for r in (_ROT_A if i % 2 == 0 else _ROT_B):
            x0 = x0 + x1
            x1 = ((x1 << jnp.uint32(r)) | (x1 >> jnp.uint32(32 - r))) ^ x0
        x0 = x0 + jnp.uint32(ks[(i + 1) % 3])
        x1 = x1 + jnp.uint32((ks[(i + 2) % 3] + i + 1) & 0xFFFFFFFF)
    return x0 ^ x1


def _bits_to_unit_float(bits):
    fb = (bits >> jnp.uint32(9)) | jnp.uint32(0x3F800000)
    return lax.bitcast_convert_type(fb, jnp.float32) - jnp.float32(1.0)


def _tc_body(pred_ref, xt_ref, t_ref, h_ref, out_ref):
    pid = pl.program_id(0)
    r0 = pid * ROWS

    row = lax.broadcasted_iota(jnp.int32, (ROWS, D), 0) + r0
    col = lax.broadcasted_iota(jnp.int32, (ROWS, D), 1)
    ctr = (row * D + col).astype(jnp.uint32)
    bits = _threefry_xor(K_SAMP, ctr)
    f = _bits_to_unit_float(bits)
    u = jnp.maximum(jnp.float32(TINY),
                    f * jnp.float32(1.0 - TINY) + jnp.float32(TINY))
    e = -jnp.log(u)

    val = pred_ref[...] - jnp.log(e)
    rowmax = jnp.max(val, axis=1, keepdims=True)
    cand = jnp.where(val == rowmax, col, jnp.int32(D))
    x1 = jnp.min(cand, axis=1)

    rctr = (lax.broadcasted_iota(jnp.int32, (ROWS,), 0) + r0).astype(jnp.uint32)
    uj = jnp.maximum(jnp.float32(0.0), _bits_to_unit_float(_threefry_xor(K_JUMP, rctr)))

    t = t_ref[0]
    h = h_ref[0]
    inten = jnp.float32(1.0) / (jnp.float32(1.0) - t)
    p_jump = jnp.float32(1.0) - jnp.exp(jnp.full((ROWS,), (-h) * inten, jnp.float32))

    xt = xt_ref[...]
    mask = (uj < p_jump) & (x1 != xt)
    out_ref[...] = jnp.where(mask, x1, xt)


def _tc_call(pred_t, t, step_size, x_t):
    return pl.pallas_call(
        _tc_body,
        grid=(N_TC // ROWS,),
        in_specs=[
            pl.BlockSpec((ROWS, D), lambda i: (i, 0)),
            pl.BlockSpec((ROWS,), lambda i: (i,)),
            pl.BlockSpec(memory_space=pltpu.SMEM),
            pl.BlockSpec(memory_space=pltpu.SMEM),
        ],
        out_specs=pl.BlockSpec((ROWS,), lambda i: (i,)),
        out_shape=jax.ShapeDtypeStruct((N_TC,), jnp.int32),
        compiler_params=pltpu.CompilerParams(
            dimension_semantics=("parallel",)),
    )(pred_t, x_t, t, step_size)


_LOGF_P = (7.0376836292e-2, -1.1514610310e-1, 1.1676998740e-1,
           -1.2420140846e-1, 1.4249322787e-1, -1.6668057665e-1,
           2.0000714765e-1, -2.4999993993e-1, 3.3333331174e-1)
_LN2 = 0.6931471805599453
_SQRTH = 0.7071067811865476


def _softlog(x):
    xi = lax.bitcast_convert_type(x, jnp.uint32)
    k = (xi >> jnp.uint32(23)).astype(jnp.int32) - jnp.int32(126)
    m = lax.bitcast_convert_type(
        (xi & jnp.uint32(0x7FFFFF)) | jnp.uint32(0x3F000000), jnp.float32)
    adj = m < jnp.float32(_SQRTH)
    m = jnp.where(adj, m + m, m)
    k = jnp.where(adj, k - jnp.int32(1), k)
    tt = m - jnp.float32(1.0)
    p = jnp.full(tt.shape, jnp.float32(_LOGF_P[0]), jnp.float32)
    for c in _LOGF_P[1:]:
        p = p * tt + jnp.float32(c)
    z = tt * tt
    y = (tt * z) * p - jnp.float32(0.5) * z
    return k.astype(jnp.float32) * jnp.float32(_LN2) + (tt + y)


def _sc_body(pred_hbm, xt_hbm, th_hbm, out_hbm, pred_v, xt_v, out_v, th_v):
    wid = lax.axis_index("s") * SC_CORES + lax.axis_index("c")
    lane = lax.iota(jnp.int32, 16)

    pltpu.sync_copy(th_hbm, th_v)
    thv = th_v[...]
    tv = jnp.full((16,), thv[0], jnp.float32)
    hv = jnp.full((16,), thv[1], jnp.float32)
    inten = jnp.float32(1.0) / (jnp.float32(1.0) - tv)
    pjv = jnp.float32(1.0) - jnp.exp((-hv) * inten)

    for b in range(SC_BLOCKS):
        grs = N_TC + wid * RW + b * BR
        ors = wid * RW + b * BR
        pltpu.sync_copy(pred_hbm.at[pl.ds(grs * D, BR * D)], pred_v)
        pltpu.sync_copy(xt_hbm.at[pl.ds(grs, BR)], xt_v)

        def group(gi, _, grs=grs):
            lrow = gi * 16 + lane
            rowv = grs + lrow
            rbase = (rowv * jnp.int32(D)).astype(jnp.uint32)

            def colstep(c, carry):
                cm, ci = carry
                colv = jnp.full((16,), c, jnp.int32)
                bits = _threefry_xor(K_SAMP, rbase + c.astype(jnp.uint32))
                f = _bits_to_unit_float(bits)
                u = jnp.maximum(jnp.float32(TINY),
                                f * jnp.float32(1.0 - TINY) + jnp.float32(TINY))
                e = -_softlog(u)
                flat = lrow * jnp.int32(D) + colv
                val = plsc.load_gather(pred_v, [flat]) - _softlog(e)
                better = val > cm
                return (jnp.where(better, val, cm),
                        jnp.where(better, colv, ci))

            cm0 = jnp.full((16,), jnp.float32(-jnp.inf), jnp.float32)
            ci0 = jnp.zeros((16,), jnp.int32)
            _, x1 = lax.fori_loop(0, D, colstep, (cm0, ci0))

            jbits = _threefry_xor(K_JUMP, rowv.astype(jnp.uint32))
            uj = jnp.maximum(jnp.float32(0.0), _bits_to_unit_float(jbits))
            xtv = xt_v[pl.ds(gi * 16, 16)]
            mask = (uj < pjv) & (x1 != xtv)
            out_v[pl.ds(gi * 16, 16)] = jnp.where(mask, x1, xtv)
            return 0

        lax.fori_loop(0, BR // 16, group, 0)
        pltpu.sync_copy(out_v, out_hbm.at[pl.ds(ors, BR)])


def _sc_call(pred_t, th, x_t):
    return pl.kernel(
        _sc_body,
        out_type=jax.ShapeDtypeStruct((N_SC,), jnp.int32),
        mesh=plsc.VectorSubcoreMesh(core_axis_name="c", subcore_axis_name="s"),
        compiler_params=pltpu.CompilerParams(needs_layout_passes=False),
        scratch_types=[
            pltpu.VMEM((BR * D,), jnp.float32),
            pltpu.VMEM((BR,), jnp.int32),
            pltpu.VMEM((BR,), jnp.int32),
            pltpu.VMEM((16,), jnp.float32),
        ],
    )(pred_t, x_t, th)


def kernel(pred_t, t, step_size, x_t):
    tc_out = _tc_call(pred_t, t, step_size, x_t)
    th = jnp.concatenate([t, step_size, jnp.zeros((14,), jnp.float32)])
    sc_out = _sc_call(pred_t.reshape(N * D), th, x_t)
    return jnp.concatenate([tc_out, sc_out])


# speedup vs baseline: 1.1625x; 1.1625x over previous
import numpy as np
import jax
import jax.numpy as jnp
from jax import lax
from jax.experimental import pallas as pl
from jax.experimental.pallas import tpu as pltpu
from jax.experimental.pallas import tpu_sc as plsc

N = 262144
D = 119
ROWS = 2048

N_SC = 16384
N_TC = N - N_SC
SC_CORES = 2
SC_SUBCORES = 16
SC_WORKERS = SC_CORES * SC_SUBCORES
RW = N_SC // SC_WORKERS
BR = 256
SC_BLOCKS = RW // BR

K_SAMP = (1832780943, 270669613)
K_JUMP = (64467757, 2916123636)
TINY = float(np.finfo(np.float32).tiny)
MAGIC = 0x1BD11BDA

_ROT_A = (13, 15, 26, 6)
_ROT_B = (17, 29, 16, 24)


def _threefry_xor(key, ctr):
    k0, k1 = int(key[0]), int(key[1])
    ks = (k0, k1, k0 ^ k1 ^ MAGIC)
    x0 = jnp.full(ctr.shape, jnp.uint32(ks[0]), jnp.uint32)
    x1 = ctr + jnp.uint32(ks[1])
    for i in range(5):
        for r in (_ROT_A if i % 2 == 0 else _ROT_B):
            x0 = x0 + x1
            x1 = ((x1 << jnp.uint32(r)) | (x1 >> jnp.uint32(32 - r))) ^ x0
        x0 = x0 + jnp.uint32(ks[(i + 1) % 3])
        x1 = x1 + jnp.uint32((ks[(i + 2) % 3] + i + 1) & 0xFFFFFFFF)
    return x0 ^ x1


def _bits_to_unit_float(bits):
    fb = (bits >> jnp.uint32(9)) | jnp.uint32(0x3F800000)
    return lax.bitcast_convert_type(fb, jnp.float32) - jnp.float32(1.0)


def _tc_body(pred_ref, xt_ref, t_ref, h_ref, out_ref):
    pid = pl.program_id(0)
    r0 = pid * ROWS

    row = lax.broadcasted_iota(jnp.int32, (ROWS, D), 0) + r0
    col = lax.broadcasted_iota(jnp.int32, (ROWS, D), 1)
    ctr = (row * D + col).astype(jnp.uint32)
    bits = _threefry_xor(K_SAMP, ctr)
    f = _bits_to_unit_float(bits)
    u = jnp.maximum(jnp.float32(TINY),
                    f * jnp.float32(1.0 - TINY) + jnp.float32(TINY))
    e = -jnp.log(u)

    val = pred_ref[...] - jnp.log(e)
    rowmax = jnp.max(val, axis=1, keepdims=True)
    cand = jnp.where(val == rowmax, col, jnp.int32(D))
    x1 = jnp.min(cand, axis=1)

    rctr = (lax.broadcasted_iota(jnp.int32, (ROWS,), 0) + r0).astype(jnp.uint32)
    uj = jnp.maximum(jnp.float32(0.0), _bits_to_unit_float(_threefry_xor(K_JUMP, rctr)))

    t = t_ref[0]
    h = h_ref[0]
    inten = jnp.float32(1.0) / (jnp.float32(1.0) - t)
    p_jump = jnp.float32(1.0) - jnp.exp(jnp.full((ROWS,), (-h) * inten, jnp.float32))

    xt = xt_ref[...]
    mask = (uj < p_jump) & (x1 != xt)
    out_ref[...] = jnp.where(mask, x1, xt)


def _tc_call(pred_t, t, step_size, x_t):
    return pl.pallas_call(
        _tc_body,
        grid=(N_TC // ROWS,),
        in_specs=[
            pl.BlockSpec((ROWS, D), lambda i: (i, 0)),
            pl.BlockSpec((ROWS,), lambda i: (i,)),
            pl.BlockSpec(memory_space=pltpu.SMEM),
            pl.BlockSpec(memory_space=pltpu.SMEM),
        ],
        out_specs=pl.BlockSpec((ROWS,), lambda i: (i,)),
        out_shape=jax.ShapeDtypeStruct((N_TC,), jnp.int32),
        compiler_params=pltpu.CompilerParams(
            dimension_semantics=("parallel",)),
    )(pred_t, x_t, t, step_size)


_LOGF_P = (7.0376836292e-2, -1.1514610310e-1, 1.1676998740e-1,
           -1.2420140846e-1, 1.4249322787e-1, -1.6668057665e-1,
           2.0000714765e-1, -2.4999993993e-1, 3.3333331174e-1)
_LN2 = 0.6931471805599453
_SQRTH = 0.7071067811865476


def _softlog(x):
    xi = lax.bitcast_convert_type(x, jnp.uint32)
    k = (xi >> jnp.uint32(23)).astype(jnp.int32) - jnp.int32(126)
    m = lax.bitcast_convert_type(
        (xi & jnp.uint32(0x7FFFFF)) | jnp.uint32(0x3F000000), jnp.float32)
    adj = m < jnp.float32(_SQRTH)
    m = jnp.where(adj, m + m, m)
    k = jnp.where(adj, k - jnp.int32(1), k)
    tt = m - jnp.float32(1.0)
    p = jnp.full(tt.shape, jnp.float32(_LOGF_P[0]), jnp.float32)
    for c in _LOGF_P[1:]:
        p = p * tt + jnp.float32(c)
    z = tt * tt
    y = (tt * z) * p - jnp.float32(0.5) * z
    return k.astype(jnp.float32) * jnp.float32(_LN2) + (tt + y)


def _sc_body(pred_hbm, xt_hbm, th_hbm, out_hbm, pred_v, xt_v, out_v, th_v):
    wid = lax.axis_index("s") * SC_CORES + lax.axis_index("c")
    lane = lax.iota(jnp.int32, 16)

    pltpu.sync_copy(th_hbm, th_v)
    thv = th_v[...]
    tv = jnp.full((16,), thv[0], jnp.float32)
    hv = jnp.full((16,), thv[1], jnp.float32)
    inten = jnp.float32(1.0) / (jnp.float32(1.0) - tv)
    pjv = jnp.float32(1.0) - jnp.exp((-hv) * inten)

    for b in range(SC_BLOCKS):
        lrs = wid * RW + b * BR
        grs = N_TC + lrs
        pltpu.sync_copy(pred_hbm.at[pl.ds(lrs * D, BR * D)], pred_v)
        pltpu.sync_copy(xt_hbm.at[pl.ds(lrs, BR)], xt_v)

        def group(gi, _, grs=grs):
            lrow = gi * 16 + lane
            rowv = grs + lrow
            rbase = (rowv * jnp.int32(D)).astype(jnp.uint32)

            def colstep(c, carry):
                cm, ci = carry
                colv = jnp.full((16,), c, jnp.int32)
                bits = _threefry_xor(K_SAMP, rbase + c.astype(jnp.uint32))
                f = _bits_to_unit_float(bits)
                u = jnp.maximum(jnp.float32(TINY),
                                f * jnp.float32(1.0 - TINY) + jnp.float32(TINY))
                e = -_softlog(u)
                flat = lrow * jnp.int32(D) + colv
                val = plsc.load_gather(pred_v, [flat]) - _softlog(e)
                better = val > cm
                return (jnp.where(better, val, cm),
                        jnp.where(better, colv, ci))

            cm0 = jnp.full((16,), jnp.float32(-jnp.inf), jnp.float32)
            ci0 = jnp.zeros((16,), jnp.int32)
            _, x1 = lax.fori_loop(0, D, colstep, (cm0, ci0))

            jbits = _threefry_xor(K_JUMP, rowv.astype(jnp.uint32))
            uj = jnp.maximum(jnp.float32(0.0), _bits_to_unit_float(jbits))
            xtv = xt_v[pl.ds(gi * 16, 16)]
            mask = (uj < pjv) & (x1 != xtv)
            out_v[pl.ds(gi * 16, 16)] = jnp.where(mask, x1, xtv)
            return 0

        lax.fori_loop(0, BR // 16, group, 0)
        pltpu.sync_copy(out_v, out_hbm.at[pl.ds(lrs, BR)])


def _sc_call(pred_t, th, x_t):
    return pl.kernel(
        _sc_body,
        out_type=jax.ShapeDtypeStruct((N_SC,), jnp.int32),
        mesh=plsc.VectorSubcoreMesh(core_axis_name="c", subcore_axis_name="s"),
        compiler_params=pltpu.CompilerParams(needs_layout_passes=False),
        scratch_types=[
            pltpu.VMEM((BR * D,), jnp.float32),
            pltpu.VMEM((BR,), jnp.int32),
            pltpu.VMEM((BR,), jnp.int32),
            pltpu.VMEM((16,), jnp.float32),
        ],
    )(pred_t, x_t, th)


def kernel(pred_t, t, step_size, x_t):
    th = jnp.concatenate([t, step_size, jnp.zeros((14,), jnp.float32)])
    sc_out = _sc_call(pred_t[N_TC:].reshape(N_SC * D), th, x_t[N_TC:])
    tc_out = _tc_call(pred_t, t, step_size, x_t)
    return jnp.concatenate([tc_out, sc_out])


# speedup vs baseline: 1.2758x; 1.0974x over previous
import numpy as np
import jax
import jax.numpy as jnp
from jax import lax
from jax.experimental import pallas as pl
from jax.experimental.pallas import tpu as pltpu
from jax.experimental.pallas import tpu_sc as plsc

N = 262144
D = 119
ROWS = 2048

N_SC = 65536
N_TC = N - N_SC
SC_CORES = 2
SC_SUBCORES = 16
SC_WORKERS = SC_CORES * SC_SUBCORES
RW = N_SC // SC_WORKERS
BR = 256
SC_BLOCKS = RW // BR

K_SAMP = (1832780943, 270669613)
K_JUMP = (64467757, 2916123636)
TINY = float(np.finfo(np.float32).tiny)
MAGIC = 0x1BD11BDA

_ROT_A = (13, 15, 26, 6)
_ROT_B = (17, 29, 16, 24)


def _threefry_xor(key, ctr):
    k0, k1 = int(key[0]), int(key[1])
    ks = (k0, k1, k0 ^ k1 ^ MAGIC)
    x0 = jnp.full(ctr.shape, jnp.uint32(ks[0]), jnp.uint32)
    x1 = ctr + jnp.uint32(ks[1])
    for i in range(5):
        for r in (_ROT_A if i % 2 == 0 else _ROT_B):
            x0 = x0 + x1
            x1 = ((x1 << jnp.uint32(r)) | (x1 >> jnp.uint32(32 - r))) ^ x0
        x0 = x0 + jnp.uint32(ks[(i + 1) % 3])
        x1 = x1 + jnp.uint32((ks[(i + 2) % 3] + i + 1) & 0xFFFFFFFF)
    return x0 ^ x1


def _bits_to_unit_float(bits):
    fb = (bits >> jnp.uint32(9)) | jnp.uint32(0x3F800000)
    return lax.bitcast_convert_type(fb, jnp.float32) - jnp.float32(1.0)


def _tc_body(pred_ref, xt_ref, t_ref, h_ref, out_ref):
    pid = pl.program_id(0)
    r0 = pid * ROWS

    row = lax.broadcasted_iota(jnp.int32, (ROWS, D), 0) + r0
    col = lax.broadcasted_iota(jnp.int32, (ROWS, D), 1)
    ctr = (row * D + col).astype(jnp.uint32)
    bits = _threefry_xor(K_SAMP, ctr)
    f = _bits_to_unit_float(bits)
    u = jnp.maximum(jnp.float32(TINY),
                    f * jnp.float32(1.0 - TINY) + jnp.float32(TINY))
    e = -jnp.log(u)

    val = pred_ref[...] - jnp.log(e)
    rowmax = jnp.max(val, axis=1, keepdims=True)
    cand = jnp.where(val == rowmax, col, jnp.int32(D))
    x1 = jnp.min(cand, axis=1)

    rctr = (lax.broadcasted_iota(jnp.int32, (ROWS,), 0) + r0).astype(jnp.uint32)
    uj = jnp.maximum(jnp.float32(0.0), _bits_to_unit_float(_threefry_xor(K_JUMP, rctr)))

    t = t_ref[0]
    h = h_ref[0]
    inten = jnp.float32(1.0) / (jnp.float32(1.0) - t)
    p_jump = jnp.float32(1.0) - jnp.exp(jnp.full((ROWS,), (-h) * inten, jnp.float32))

    xt = xt_ref[...]
    mask = (uj < p_jump) & (x1 != xt)
    out_ref[...] = jnp.where(mask, x1, xt)


def _tc_call(pred_t, t, step_size, x_t):
    return pl.pallas_call(
        _tc_body,
        grid=(N_TC // ROWS,),
        in_specs=[
            pl.BlockSpec((ROWS, D), lambda i: (i, 0)),
            pl.BlockSpec((ROWS,), lambda i: (i,)),
            pl.BlockSpec(memory_space=pltpu.SMEM),
            pl.BlockSpec(memory_space=pltpu.SMEM),
        ],
        out_specs=pl.BlockSpec((ROWS,), lambda i: (i,)),
        out_shape=jax.ShapeDtypeStruct((N_TC,), jnp.int32),
        compiler_params=pltpu.CompilerParams(
            dimension_semantics=("parallel",)),
    )(pred_t, x_t, t, step_size)


_LOGF_P = (7.0376836292e-2, -1.1514610310e-1, 1.1676998740e-1,
           -1.2420140846e-1, 1.4249322787e-1, -1.6668057665e-1,
           2.0000714765e-1, -2.4999993993e-1, 3.3333331174e-1)
_LN2 = 0.6931471805599453
_SQRTH = 0.7071067811865476


def _softlog(x):
    xi = lax.bitcast_convert_type(x, jnp.uint32)
    k = (xi >> jnp.uint32(23)).astype(jnp.int32) - jnp.int32(126)
    m = lax.bitcast_convert_type(
        (xi & jnp.uint32(0x7FFFFF)) | jnp.uint32(0x3F000000), jnp.float32)
    adj = m < jnp.float32(_SQRTH)
    m = jnp.where(adj, m + m, m)
    k = jnp.where(adj, k - jnp.int32(1), k)
    tt = m - jnp.float32(1.0)
    p = jnp.full(tt.shape, jnp.float32(_LOGF_P[0]), jnp.float32)
    for c in _LOGF_P[1:]:
        p = p * tt + jnp.float32(c)
    z = tt * tt
    y = (tt * z) * p - jnp.float32(0.5) * z
    return k.astype(jnp.float32) * jnp.float32(_LN2) + (tt + y)


def _sc_body(pred_hbm, xt_hbm, th_hbm, out_hbm, pred_v, xt_v, out_v, th_v):
    wid = lax.axis_index("s") * SC_CORES + lax.axis_index("c")
    lane = lax.iota(jnp.int32, 16)

    pltpu.sync_copy(th_hbm, th_v)
    thv = th_v[...]
    tv = jnp.full((16,), thv[0], jnp.float32)
    hv = jnp.full((16,), thv[1], jnp.float32)
    inten = jnp.float32(1.0) / (jnp.float32(1.0) - tv)
    pjv = jnp.float32(1.0) - jnp.exp((-hv) * inten)

    for b in range(SC_BLOCKS):
        lrs = wid * RW + b * BR
        grs = N_TC + lrs
        pltpu.sync_copy(pred_hbm.at[pl.ds(lrs * D, BR * D)], pred_v)
        pltpu.sync_copy(xt_hbm.at[pl.ds(lrs, BR)], xt_v)

        def group(gi, _, grs=grs):
            lrow = gi * 16 + lane
            rowv = grs + lrow
            rbase = (rowv * jnp.int32(D)).astype(jnp.uint32)

            def colstep(c, carry):
                cm, ci = carry
                colv = jnp.full((16,), c, jnp.int32)
                bits = _threefry_xor(K_SAMP, rbase + c.astype(jnp.uint32))
                f = _bits_to_unit_float(bits)
                u = jnp.maximum(jnp.float32(TINY),
                                f * jnp.float32(1.0 - TINY) + jnp.float32(TINY))
                e = -_softlog(u)
                flat = lrow * jnp.int32(D) + colv
                val = plsc.load_gather(pred_v, [flat]) - _softlog(e)
                better = val > cm
                return (jnp.where(better, val, cm),
                        jnp.where(better, colv, ci))

            cm0 = jnp.full((16,), jnp.float32(-jnp.inf), jnp.float32)
            ci0 = jnp.zeros((16,), jnp.int32)
            _, x1 = lax.fori_loop(0, D, colstep, (cm0, ci0))

            jbits = _threefry_xor(K_JUMP, rowv.astype(jnp.uint32))
            uj = jnp.maximum(jnp.float32(0.0), _bits_to_unit_float(jbits))
            xtv = xt_v[pl.ds(gi * 16, 16)]
            mask = (uj < pjv) & (x1 != xtv)
            out_v[pl.ds(gi * 16, 16)] = jnp.where(mask, x1, xtv)
            return 0

        lax.fori_loop(0, BR // 16, group, 0)
        pltpu.sync_copy(out_v, out_hbm.at[pl.ds(lrs, BR)])


def _sc_call(pred_t, th, x_t):
    return pl.kernel(
        _sc_body,
        out_type=jax.ShapeDtypeStruct((N_SC,), jnp.int32),
        mesh=plsc.VectorSubcoreMesh(core_axis_name="c", subcore_axis_name="s"),
        compiler_params=pltpu.CompilerParams(needs_layout_passes=False),
        scratch_types=[
            pltpu.VMEM((BR * D,), jnp.float32),
            pltpu.VMEM((BR,), jnp.int32),
            pltpu.VMEM((BR,), jnp.int32),
            pltpu.VMEM((16,), jnp.float32),
        ],
    )(pred_t, x_t, th)


def kernel(pred_t, t, step_size, x_t):
    th = jnp.concatenate([t, step_size, jnp.zeros((14,), jnp.float32)])
    sc_out = _sc_call(pred_t[N_TC:].reshape(N_SC * D), th, x_t[N_TC:])
    tc_out = _tc_call(pred_t, t, step_size, x_t)
    return jnp.concatenate([tc_out, sc_out])
